# baseline (speedup 1.0000x reference)
import jax
import jax.numpy as jnp
from jax.experimental import pallas as pl
from jax.experimental.pallas import tpu as pltpu

_D = 64
_K = 1024
_B = 16
_HW = 1024
_N = _B * _HW
_COMMIT = 0.25


def _vq_body(x_ref, embt_ref, emb_ref,
             dist_ref, enc_ref, q_ref, idx_ref, loss_ref, perp_ref,
             e2_ref, iota_ref, perm_ref, cnt_ref, sse_ref):
    i = pl.program_id(0)
    embt = embt_ref[...]

    @pl.when(i == 0)
    def _():
        e2_ref[...] = jnp.sum(embt * embt, axis=0, keepdims=True)
        iota_ref[...] = jax.lax.broadcasted_iota(
            jnp.int32, (_HW, _K), 1).astype(jnp.float32)
        r = jax.lax.broadcasted_iota(jnp.int32, (_HW, _HW), 0)
        c = jax.lax.broadcasted_iota(jnp.int32, (_HW, _HW), 1)
        perm_ref[...] = ((r % 32) * 32 + r // 32 == c).astype(jnp.float32)

    perm = perm_ref[...]
    xc = x_ref[...]
    x = jax.lax.dot_general(perm, xc, (((1,), (1,)), ((), ())),
                            preferred_element_type=jnp.float32)
    x2 = jnp.sum(x * x, axis=1, keepdims=True)
    mm = jnp.matmul(x, embt, preferred_element_type=jnp.float32)
    dist = (x2 - 2.0 * mm) + e2_ref[...]
    dist_ref[...] = dist

    mind = jnp.min(dist, axis=1, keepdims=True)
    iota = iota_ref[...]
    idx_f = jnp.min(jnp.where(dist == mind, iota, float(_K)), axis=1,
                    keepdims=True)
    idx_row = idx_f.astype(jnp.int32).reshape(1, 1, _HW)
    idx_ref[...] = idx_row
    enc = (iota == idx_f).astype(jnp.float32)
    enc_ref[...] = enc
    qm = jnp.matmul(enc, emb_ref[...],
                    preferred_element_type=jnp.float32)
    qc = jax.lax.dot_general(qm, perm, (((0,), (0,)), ((), ())),
                             preferred_element_type=jnp.float32)
    q_ref[...] = qc

    ssum = jnp.sum(mind, keepdims=True)
    cnt = jnp.sum(enc, axis=0, keepdims=True)

    @pl.when(i == 0)
    def _():
        cnt_ref[...] = cnt
        sse_ref[...] = ssum

    @pl.when(i > 0)
    def _():
        cnt_ref[...] += cnt
        sse_ref[...] += ssum

    @pl.when(i == _B - 1)
    def _():
        loss_ref[...] = (1.0 + _COMMIT) * (sse_ref[...] / (_N * _D))
        avg = cnt_ref[...] / _N
        ent = jnp.sum(avg * jnp.log(avg + 1e-10), keepdims=True)
        perp_ref[...] = jnp.exp(-ent)


def kernel(inputs, is_training, embeddings):
    inp2 = inputs.reshape(_B * _D, _HW)
    embt = embeddings.T

    dist, enc, q3, idx3, loss, perp = pl.pallas_call(
        _vq_body,
        grid=(_B,),
        in_specs=[
            pl.BlockSpec((_D, _HW), lambda i: (i, 0)),
            pl.BlockSpec((_D, _K), lambda i: (0, 0)),
            pl.BlockSpec((_K, _D), lambda i: (0, 0)),
        ],
        out_specs=[
            pl.BlockSpec((_HW, _K), lambda i: (i, 0)),
            pl.BlockSpec((_HW, _K), lambda i: (i, 0)),
            pl.BlockSpec((_D, _HW), lambda i: (i, 0)),
            pl.BlockSpec((1, 1, _HW), lambda i: (i, 0, 0)),
            pl.BlockSpec((1, 1), lambda i: (0, 0)),
            pl.BlockSpec((1, 1), lambda i: (0, 0)),
        ],
        out_shape=[
            jax.ShapeDtypeStruct((_N, _K), jnp.float32),
            jax.ShapeDtypeStruct((_N, _K), jnp.float32),
            jax.ShapeDtypeStruct((_B * _D, _HW), jnp.float32),
            jax.ShapeDtypeStruct((_B, 1, _HW), jnp.int32),
            jax.ShapeDtypeStruct((1, 1), jnp.float32),
            jax.ShapeDtypeStruct((1, 1), jnp.float32),
        ],
        scratch_shapes=[
            pltpu.VMEM((1, _K), jnp.float32),
            pltpu.VMEM((_HW, _K), jnp.float32),
            pltpu.VMEM((_HW, _HW), jnp.float32),
            pltpu.VMEM((1, _K), jnp.float32),
            pltpu.VMEM((1, 1), jnp.float32),
        ],
    )(inp2, embt, embeddings)

    enc_idx = idx3.reshape(16, 32, 32)
    quantize = q3.reshape(16, _D, 32, 32)
    return (quantize, loss[0, 0], perp[0, 0], enc, enc_idx, dist)


# speedup vs baseline: 1.3680x; 1.3680x over previous
import jax
import jax.numpy as jnp
from jax.experimental import pallas as pl
from jax.experimental.pallas import tpu as pltpu

_D = 64
_K = 1024
_N = 16 * 32 * 32
_TILE_M = 1024
_NT = _N // _TILE_M
_COMMIT = 0.25


def _vq_body(x_ref, embt_ref, emb_ref,
             dist_ref, enc_ref, q_ref, idx_ref, loss_ref, perp_ref,
             e2_ref, iota_ref, cnt_ref, sse_ref):
    i = pl.program_id(0)
    embt = embt_ref[...]

    @pl.when(i == 0)
    def _():
        e2_ref[...] = jnp.sum(embt * embt, axis=0, keepdims=True)
        iota_ref[...] = jax.lax.broadcasted_iota(
            jnp.int32, (_TILE_M, _K), 1).astype(jnp.float32)

    x = x_ref[...]
    x2 = jnp.sum(x * x, axis=1, keepdims=True)
    mm = jnp.matmul(x, embt, preferred_element_type=jnp.float32)
    dist = (x2 - 2.0 * mm) + e2_ref[...]
    dist_ref[...] = dist

    mind = jnp.min(dist, axis=1, keepdims=True)
    iota = iota_ref[...]
    idx_f = jnp.min(jnp.where(dist == mind, iota, float(_K)), axis=1,
                    keepdims=True)
    idx_ref[...] = idx_f.astype(jnp.int32).reshape(1, 1, _TILE_M)
    enc = (iota == idx_f).astype(jnp.float32)
    enc_ref[...] = enc
    q_ref[...] = jnp.matmul(enc, emb_ref[...],
                            preferred_element_type=jnp.float32)

    ssum = jnp.sum(mind, keepdims=True)
    cnt = jnp.sum(enc, axis=0, keepdims=True)

    @pl.when(i == 0)
    def _():
        cnt_ref[...] = cnt
        sse_ref[...] = ssum

    @pl.when(i > 0)
    def _():
        cnt_ref[...] += cnt
        sse_ref[...] += ssum

    @pl.when(i == _NT - 1)
    def _():
        loss_ref[...] = (1.0 + _COMMIT) * (sse_ref[...] / (_N * _D))
        avg = cnt_ref[...] / _N
        ent = jnp.sum(avg * jnp.log(avg + 1e-10), keepdims=True)
        perp_ref[...] = jnp.exp(-ent)


def kernel(inputs, is_training, embeddings):
    x = jnp.transpose(inputs, (0, 3, 2, 1))
    flat = x.reshape(_N, _D)
    embt = embeddings.T

    dist, enc, q, idx3, loss, perp = pl.pallas_call(
        _vq_body,
        grid=(_NT,),
        in_specs=[
            pl.BlockSpec((_TILE_M, _D), lambda i: (i, 0)),
            pl.BlockSpec((_D, _K), lambda i: (0, 0)),
            pl.BlockSpec((_K, _D), lambda i: (0, 0)),
        ],
        out_specs=[
            pl.BlockSpec((_TILE_M, _K), lambda i: (i, 0)),
            pl.BlockSpec((_TILE_M, _K), lambda i: (i, 0)),
            pl.BlockSpec((_TILE_M, _D), lambda i: (i, 0)),
            pl.BlockSpec((1, 1, _TILE_M), lambda i: (i, 0, 0)),
            pl.BlockSpec((1, 1), lambda i: (0, 0)),
            pl.BlockSpec((1, 1), lambda i: (0, 0)),
        ],
        out_shape=[
            jax.ShapeDtypeStruct((_N, _K), jnp.float32),
            jax.ShapeDtypeStruct((_N, _K), jnp.float32),
            jax.ShapeDtypeStruct((_N, _D), jnp.float32),
            jax.ShapeDtypeStruct((_NT, 1, _TILE_M), jnp.int32),
            jax.ShapeDtypeStruct((1, 1), jnp.float32),
            jax.ShapeDtypeStruct((1, 1), jnp.float32),
        ],
        scratch_shapes=[
            pltpu.VMEM((1, _K), jnp.float32),
            pltpu.VMEM((_TILE_M, _K), jnp.float32),
            pltpu.VMEM((1, _K), jnp.float32),
            pltpu.VMEM((1, 1), jnp.float32),
        ],
    )(flat, embt, embeddings)

    enc_idx = idx3.reshape(16, 32, 32)
    quantize = jnp.transpose(q.reshape(16, 32, 32, _D), (0, 3, 2, 1))
    return (quantize, loss[0, 0], perp[0, 0], enc, enc_idx, dist)
